# baseline (speedup 1.0000x reference)
import jax
import jax.numpy as jnp
from jax import lax
from jax.experimental import pallas as pl
from jax.experimental.pallas import tpu as pltpu
from jax.experimental.pallas import tpu_sc as plsc

_EMBED_DIM = 64
_BATCH = 16384

_info = plsc.get_sparse_core_info()
_NC, _NS, _L = _info.num_cores, _info.num_subcores, _info.num_lanes
_NW = _NC * _NS
_BPW = _BATCH // _NW
_GROUPS = _BPW // _L


def _sc_body(u_idx_hbm, i_idx_hbm, user_hbm, item_hbm, out_hbm,
             uidx_v, iidx_v, urows_v, irows_v, accbuf_v, out_v, sem_u, sem_i):
    wid = lax.axis_index("s") * _NC + lax.axis_index("c")
    base = wid * _BPW
    pltpu.sync_copy(u_idx_hbm.at[pl.ds(base, _BPW)], uidx_v)
    pltpu.sync_copy(i_idx_hbm.at[pl.ds(base, _BPW)], iidx_v)
    cu = pltpu.async_copy(user_hbm.at[uidx_v], urows_v, sem_u)
    ci = pltpu.async_copy(item_hbm.at[iidx_v], irows_v, sem_i)
    cu.wait()
    ci.wait()

    lanes = lax.iota(jnp.int32, _L)

    def group(g, carry):
        for j in range(_L):
            r = g * _L + j
            acc = jnp.zeros((_L,), jnp.float32)
            for c in range(_EMBED_DIM // _L):
                u = urows_v[r, pl.ds(c * _L, _L)]
                v = irows_v[r, pl.ds(c * _L, _L)]
                acc = acc + u * v
            accbuf_v[pl.ds(j * _L, _L)] = acc
        s = jnp.zeros((_L,), jnp.float32)
        for c in range(_L):
            col = plsc.load_gather(accbuf_v, [lanes * _L + c])
            s = s + col
        out_v[pl.ds(g * _L, _L)] = s
        return carry

    lax.fori_loop(0, _GROUPS, group, 0)
    pltpu.sync_copy(out_v, out_hbm.at[pl.ds(base, _BPW)])


@jax.jit
def _run(user_idx, item_idx, user_table, item_table):
    mesh = plsc.VectorSubcoreMesh(core_axis_name="c", subcore_axis_name="s")
    call = pl.kernel(
        _sc_body,
        out_type=jax.ShapeDtypeStruct((_BATCH,), jnp.float32),
        mesh=mesh,
        compiler_params=pltpu.CompilerParams(
            needs_layout_passes=False, use_tc_tiling_on_sc=False),
        scratch_types=[
            pltpu.VMEM((_BPW,), jnp.int32),
            pltpu.VMEM((_BPW,), jnp.int32),
            pltpu.VMEM((_BPW, _EMBED_DIM), jnp.float32),
            pltpu.VMEM((_BPW, _EMBED_DIM), jnp.float32),
            pltpu.VMEM((_L * _L,), jnp.float32),
            pltpu.VMEM((_BPW,), jnp.float32),
            pltpu.SemaphoreType.DMA,
            pltpu.SemaphoreType.DMA,
        ],
    )
    return call(user_idx, item_idx, user_table, item_table)


def kernel(user_inputs, item_inputs, user_table, item_table):
    y = _run(user_inputs.reshape(-1), item_inputs.reshape(-1),
             user_table, item_table)
    return y.reshape(_BATCH, 1)


# speedup vs baseline: 1.5440x; 1.5440x over previous
import jax
import jax.numpy as jnp
from jax import lax
from jax.experimental import pallas as pl
from jax.experimental.pallas import tpu as pltpu
from jax.experimental.pallas import tpu_sc as plsc

_EMBED_DIM = 64
_BATCH = 16384

_info = plsc.get_sparse_core_info()
_NC, _NS, _L = _info.num_cores, _info.num_subcores, _info.num_lanes
_NW = _NC * _NS
_BPW = _BATCH // _NW
_CHUNK = 64
_NCH = _BPW // _CHUNK
_CGROUPS = _CHUNK // _L


def _sc_body(u_idx_hbm, i_idx_hbm, user_hbm, item_hbm, out_hbm,
             uidx_v, iidx_v, ubuf0, ubuf1, ibuf0, ibuf1,
             accbuf_v, out_v, sem_u0, sem_u1, sem_i0, sem_i1):
    wid = lax.axis_index("s") * _NC + lax.axis_index("c")
    base = wid * _BPW
    pltpu.sync_copy(u_idx_hbm.at[pl.ds(base, _BPW)], uidx_v)
    pltpu.sync_copy(i_idx_hbm.at[pl.ds(base, _BPW)], iidx_v)

    ubufs = (ubuf0, ubuf1)
    ibufs = (ibuf0, ibuf1)
    usems = (sem_u0, sem_u1)
    isems = (sem_i0, sem_i1)
    lanes = lax.iota(jnp.int32, _L)

    def fire(c, b):
        for q in range(_CHUNK // _L):
            uvec = uidx_v[pl.ds(c * _CHUNK + q * _L, _L)]
            ivec = iidx_v[pl.ds(c * _CHUNK + q * _L, _L)]
            for j in range(_L):
                row = q * _L + j
                pltpu.async_copy(user_hbm.at[uvec[j]], ubufs[b].at[row], usems[b])
                pltpu.async_copy(item_hbm.at[ivec[j]], ibufs[b].at[row], isems[b])

    def drain(b):
        for j in range(_CHUNK):
            pltpu.make_async_copy(user_hbm.at[0], ubufs[b].at[j], usems[b]).wait()
            pltpu.make_async_copy(item_hbm.at[0], ibufs[b].at[j], isems[b]).wait()

    def compute(c, b):
        for g in range(_CGROUPS):
            for j in range(_L):
                row = g * _L + j
                acc = jnp.zeros((_L,), jnp.float32)
                for k in range(_EMBED_DIM // _L):
                    u = ubufs[b][row, pl.ds(k * _L, _L)]
                    v = ibufs[b][row, pl.ds(k * _L, _L)]
                    acc = acc + u * v
                accbuf_v[pl.ds(j * _L, _L)] = acc
            s = jnp.zeros((_L,), jnp.float32)
            for col in range(_L):
                s = s + plsc.load_gather(accbuf_v, [lanes * _L + col])
            out_v[pl.ds(c * _CHUNK + g * _L, _L)] = s

    fire(0, 0)
    fire(1, 1)

    def step(s, carry):
        for b in range(2):
            c = 2 * s + b
            drain(b)
            compute(c, b)

            @pl.when(c + 2 < _NCH)
            def _():
                fire(c + 2, b)
        return carry

    lax.fori_loop(0, _NCH // 2, step, 0)
    pltpu.sync_copy(out_v, out_hbm.at[pl.ds(base, _BPW)])


@jax.jit
def _run(user_idx, item_idx, user_table, item_table):
    mesh = plsc.VectorSubcoreMesh(core_axis_name="c", subcore_axis_name="s")
    call = pl.kernel(
        _sc_body,
        out_type=jax.ShapeDtypeStruct((_BATCH,), jnp.float32),
        mesh=mesh,
        compiler_params=pltpu.CompilerParams(needs_layout_passes=False),
        scratch_types=[
            pltpu.VMEM((_BPW,), jnp.int32),
            pltpu.VMEM((_BPW,), jnp.int32),
            pltpu.VMEM((_CHUNK, _EMBED_DIM), jnp.float32),
            pltpu.VMEM((_CHUNK, _EMBED_DIM), jnp.float32),
            pltpu.VMEM((_CHUNK, _EMBED_DIM), jnp.float32),
            pltpu.VMEM((_CHUNK, _EMBED_DIM), jnp.float32),
            pltpu.VMEM((_L * _L,), jnp.float32),
            pltpu.VMEM((_BPW,), jnp.float32),
            pltpu.SemaphoreType.DMA,
            pltpu.SemaphoreType.DMA,
            pltpu.SemaphoreType.DMA,
            pltpu.SemaphoreType.DMA,
        ],
    )
    return call(user_idx, item_idx, user_table, item_table)


def kernel(user_inputs, item_inputs, user_table, item_table):
    y = _run(user_inputs.reshape(-1), item_inputs.reshape(-1),
             user_table, item_table)
    return y.reshape(_BATCH, 1)


# speedup vs baseline: 3.6059x; 2.3354x over previous
import functools

import jax
import jax.numpy as jnp
from jax import lax
from jax.experimental import pallas as pl
from jax.experimental.pallas import tpu as pltpu
from jax.experimental.pallas import tpu_sc as plsc

_D = 64
_BATCH = 16384
_V = 1000000

_info = plsc.get_sparse_core_info()
_NC, _NS, _L = _info.num_cores, _info.num_subcores, _info.num_lanes
_NW = _NC * _NS

_TPC = (_V + 127) // 128
_WT = (_TPC + _NS - 1) // _NS
_CT = 4
_SLABW = _CT * 128
_NCHUNK = (_WT + _CT - 1) // _CT
_MAXCT = _TPC - _CT
_WCAP = 2560
_CCAP = 128

_BPW = _BATCH // _NW
_RCH = 128


def _extract_pipeline(idx_hbm, table_hbm, rows_hbm, idx_v, slab0, slab1,
                      ulist, blist, culist, cblist, rowstage, sub,
                      sem_s0, sem_s1, sem_r):
    slabs = (slab0, slab1)
    sems = (sem_s0, sem_s1)
    wlo = sub * (_WT * 128)
    whi = wlo + _WT * 128
    lanes = lax.iota(jnp.int32, _L)

    pltpu.sync_copy(idx_hbm, idx_v)

    def scan_group(g, cnt):
        uvec = idx_v[pl.ds(g * _L, _L)]
        bvec = g * _L + lanes
        mask = (uvec >= wlo) & (uvec < whi)
        plsc.store_compressed(ulist.at[pl.ds(cnt, _L)], uvec, mask=mask)
        plsc.store_compressed(blist.at[pl.ds(cnt, _L)], bvec, mask=mask)
        c16 = plsc.all_reduce_population_count(mask)
        return cnt + c16[0]

    cnt = lax.fori_loop(0, _BATCH // _L, scan_group, jnp.int32(0))
    nsel_groups = (cnt + _L - 1) // _L

    def slab_start(k):
        ct = jnp.minimum(sub * _WT + k * _CT, _MAXCT)
        return pl.multiple_of(ct * 128, 128)

    def fire(k, b):
        pltpu.async_copy(table_hbm.at[:, pl.ds(slab_start(k), _SLABW)],
                         slabs[b], sems[b])

    def drain(b):
        pltpu.make_async_copy(
            table_hbm.at[:, pl.ds(pl.multiple_of(jnp.int32(0), 128), _SLABW)],
            slabs[b], sems[b]).wait()

    def extract(k, b):
        clo = slab_start(k)
        chi = jnp.minimum(clo + _SLABW, whi)

        def match_group(g, ccnt):
            uvec = ulist[pl.ds(g * _L, _L)]
            bvec = blist[pl.ds(g * _L, _L)]
            valid = (g * _L + lanes) < cnt
            mask = valid & (uvec >= clo) & (uvec < chi)
            plsc.store_compressed(culist.at[pl.ds(ccnt, _L)], uvec - clo,
                                  mask=mask)
            plsc.store_compressed(cblist.at[pl.ds(ccnt, _L)], bvec, mask=mask)
            c16 = plsc.all_reduce_population_count(mask)
            return ccnt + c16[0]

        ccnt = lax.fori_loop(0, nsel_groups, match_group, jnp.int32(0))

        def out_group(g2, carry):
            cu = culist[pl.ds(g2 * _L, _L)]
            cb = cblist[pl.ds(g2 * _L, _L)]
            for j in range(_L):
                @pl.when(g2 * _L + j < ccnt)
                def _():
                    colv = jnp.full((_L,), cu[j], jnp.int32)
                    for dg in range(_D // _L):
                        col = plsc.load_gather(slabs[b],
                                               [dg * _L + lanes, colv])
                        rowstage[j, pl.ds(dg * _L, _L)] = col
                    pltpu.async_copy(rowstage.at[j], rows_hbm.at[cb[j]],
                                     sem_r)
            for j in range(_L):
                @pl.when(g2 * _L + j < ccnt)
                def _():
                    pltpu.make_async_copy(rowstage.at[j], rows_hbm.at[cb[j]],
                                          sem_r).wait()
            return carry

        lax.fori_loop(0, (ccnt + _L - 1) // _L, out_group, jnp.int32(0))

    fire(0, 0)
    fire(1, 1)
    total_chunks = 2 * ((_NCHUNK + 1) // 2)

    def step(si, carry):
        for b in range(2):
            k = 2 * si + b
            drain(b)
            extract(k, b)

            @pl.when(k + 2 < total_chunks)
            def _():
                fire(k + 2, b)
        return carry

    lax.fori_loop(0, total_chunks // 2, step, jnp.int32(0))


def _extract_body(u_idx_hbm, i_idx_hbm, ut_hbm, it_hbm, urows_hbm, irows_hbm,
                  idx_v, slab0, slab1, ulist, blist, culist, cblist, rowstage,
                  sem_s0, sem_s1, sem_r):
    c = lax.axis_index("c")
    sub = lax.axis_index("s")

    @pl.when(c == 0)
    def _():
        _extract_pipeline(u_idx_hbm, ut_hbm, urows_hbm, idx_v, slab0, slab1,
                          ulist, blist, culist, cblist, rowstage, sub,
                          sem_s0, sem_s1, sem_r)

    @pl.when(c == 1)
    def _():
        _extract_pipeline(i_idx_hbm, it_hbm, irows_hbm, idx_v, slab0, slab1,
                          ulist, blist, culist, cblist, rowstage, sub,
                          sem_s0, sem_s1, sem_r)


def _dot_body(urows_hbm, irows_hbm, out_hbm,
              ubuf0, ubuf1, ibuf0, ibuf1, out_v,
              sem_u0, sem_u1, sem_i0, sem_i1):
    wid = lax.axis_index("s") * _NC + lax.axis_index("c")
    base = wid * _BPW
    ubufs, ibufs = (ubuf0, ubuf1), (ibuf0, ibuf1)
    usems, isems = (sem_u0, sem_u1), (sem_i0, sem_i1)
    lanes = lax.iota(jnp.int32, _L)

    def fire(k, b):
        r0 = base + k * _RCH
        pltpu.async_copy(urows_hbm.at[pl.ds(r0, _RCH), :], ubufs[b], usems[b])
        pltpu.async_copy(irows_hbm.at[pl.ds(r0, _RCH), :], ibufs[b], isems[b])

    def drain(b):
        pltpu.make_async_copy(urows_hbm.at[pl.ds(0, _RCH), :], ubufs[b],
                              usems[b]).wait()
        pltpu.make_async_copy(irows_hbm.at[pl.ds(0, _RCH), :], ibufs[b],
                              isems[b]).wait()

    def compute(k, b):
        for g in range(_RCH // _L):
            rows = g * _L + lanes
            acc = jnp.zeros((_L,), jnp.float32)
            for d in range(_D):
                dcol = jnp.full((_L,), d, jnp.int32)
                u = plsc.load_gather(ubufs[b], [rows, dcol])
                v = plsc.load_gather(ibufs[b], [rows, dcol])
                acc = acc + u * v
            out_v[pl.ds(k * _RCH + g * _L, _L)] = acc

    fire(0, 0)
    fire(1, 1)
    nch = _BPW // _RCH
    for k in range(nch):
        b = k % 2
        drain(b)
        compute(k, b)
        if k + 2 < nch:
            fire(k + 2, b)
    pltpu.sync_copy(out_v, out_hbm.at[pl.ds(base, _BPW)])


@jax.jit
def _run(user_idx, item_idx, user_table_t, item_table_t):
    mesh = plsc.VectorSubcoreMesh(core_axis_name="c", subcore_axis_name="s")
    params = pltpu.CompilerParams(needs_layout_passes=False)

    extract = pl.kernel(
        _extract_body,
        out_type=(jax.ShapeDtypeStruct((_BATCH, _D), jnp.float32),
                  jax.ShapeDtypeStruct((_BATCH, _D), jnp.float32)),
        mesh=mesh,
        compiler_params=params,
        scratch_types=[
            pltpu.VMEM((_BATCH,), jnp.int32),
            pltpu.VMEM((_D, _SLABW), jnp.float32),
            pltpu.VMEM((_D, _SLABW), jnp.float32),
            pltpu.VMEM((_WCAP,), jnp.int32),
            pltpu.VMEM((_WCAP,), jnp.int32),
            pltpu.VMEM((_CCAP,), jnp.int32),
            pltpu.VMEM((_CCAP,), jnp.int32),
            pltpu.VMEM((_L, _D), jnp.float32),
            pltpu.SemaphoreType.DMA,
            pltpu.SemaphoreType.DMA,
            pltpu.SemaphoreType.DMA,
        ],
    )
    urows, irows = extract(user_idx, item_idx, user_table_t, item_table_t)

    dot = pl.kernel(
        _dot_body,
        out_type=jax.ShapeDtypeStruct((_BATCH,), jnp.float32),
        mesh=mesh,
        compiler_params=params,
        scratch_types=[
            pltpu.VMEM((_RCH, _D), jnp.float32),
            pltpu.VMEM((_RCH, _D), jnp.float32),
            pltpu.VMEM((_RCH, _D), jnp.float32),
            pltpu.VMEM((_RCH, _D), jnp.float32),
            pltpu.VMEM((_BPW,), jnp.float32),
            pltpu.SemaphoreType.DMA,
            pltpu.SemaphoreType.DMA,
            pltpu.SemaphoreType.DMA,
            pltpu.SemaphoreType.DMA,
        ],
    )
    return dot(urows, irows)


def kernel(user_inputs, item_inputs, user_table, item_table):
    y = _run(user_inputs.reshape(-1), item_inputs.reshape(-1),
             user_table.T, item_table.T)
    return y.reshape(_BATCH, 1)


# speedup vs baseline: 4.1435x; 1.1491x over previous
import functools

import jax
import jax.numpy as jnp
from jax import lax
from jax.experimental import pallas as pl
from jax.experimental.pallas import tpu as pltpu
from jax.experimental.pallas import tpu_sc as plsc

_D = 64
_BATCH = 16384
_V = 1000000

_info = plsc.get_sparse_core_info()
_NC, _NS, _L = _info.num_cores, _info.num_subcores, _info.num_lanes
_NW = _NC * _NS

_TPC = (_V + 127) // 128
_WT = (_TPC + _NS - 1) // _NS
_CT = 6
_SLABW = _CT * 128
_NCHUNK = (_WT + _CT - 1) // _CT
_MAXCT = _TPC - _CT
_WCAP = 2560
_CCAP = 128

_BPW = _BATCH // _NW
_RCH = 128


def _extract_pipeline(idx_hbm, table_hbm, rows_hbm, idx_v, slab0, slab1,
                      ulist, blist, culist, cblist, rowstage, sub,
                      sem_s0, sem_s1, sem_r):
    slabs = (slab0, slab1)
    sems = (sem_s0, sem_s1)
    wlo = sub * (_WT * 128)
    whi = wlo + _WT * 128
    lanes = lax.iota(jnp.int32, _L)

    pltpu.sync_copy(idx_hbm, idx_v)

    def scan_group(g, cnt):
        uvec = idx_v[pl.ds(g * _L, _L)]
        bvec = g * _L + lanes
        mask = (uvec >= wlo) & (uvec < whi)
        plsc.store_compressed(ulist.at[pl.ds(cnt, _L)], uvec, mask=mask)
        plsc.store_compressed(blist.at[pl.ds(cnt, _L)], bvec, mask=mask)
        c16 = plsc.all_reduce_population_count(mask)
        return cnt + c16[0]

    cnt = lax.fori_loop(0, _BATCH // _L, scan_group, jnp.int32(0))
    nsel_groups = (cnt + _L - 1) // _L

    def slab_start(k):
        ct = jnp.minimum(sub * _WT + k * _CT, _MAXCT)
        return pl.multiple_of(ct * 128, 128)

    def fire(k, b):
        pltpu.async_copy(table_hbm.at[:, pl.ds(slab_start(k), _SLABW)],
                         slabs[b], sems[b])

    def drain(b):
        pltpu.make_async_copy(
            table_hbm.at[:, pl.ds(pl.multiple_of(jnp.int32(0), 128), _SLABW)],
            slabs[b], sems[b]).wait()

    def extract(k, b):
        clo = slab_start(k)
        chi = jnp.minimum(clo + _SLABW, whi)

        def match_group(g, ccnt):
            uvec = ulist[pl.ds(g * _L, _L)]
            bvec = blist[pl.ds(g * _L, _L)]
            valid = (g * _L + lanes) < cnt
            mask = valid & (uvec >= clo) & (uvec < chi)
            plsc.store_compressed(culist.at[pl.ds(ccnt, _L)], uvec - clo,
                                  mask=mask)
            plsc.store_compressed(cblist.at[pl.ds(ccnt, _L)], bvec, mask=mask)
            c16 = plsc.all_reduce_population_count(mask)
            return ccnt + c16[0]

        ccnt = lax.fori_loop(0, nsel_groups, match_group, jnp.int32(0))

        def out_group(g2, carry):
            cu = culist[pl.ds(g2 * _L, _L)]
            cb = cblist[pl.ds(g2 * _L, _L)]
            for j in range(_L):
                @pl.when(g2 * _L + j < ccnt)
                def _():
                    colv = jnp.full((_L,), cu[j], jnp.int32)
                    for dg in range(_D // _L):
                        col = plsc.load_gather(slabs[b],
                                               [dg * _L + lanes, colv])
                        rowstage[j, pl.ds(dg * _L, _L)] = col
                    pltpu.async_copy(rowstage.at[j], rows_hbm.at[cb[j]],
                                     sem_r)
            for j in range(_L):
                @pl.when(g2 * _L + j < ccnt)
                def _():
                    pltpu.make_async_copy(rowstage.at[j], rows_hbm.at[cb[j]],
                                          sem_r).wait()
            return carry

        lax.fori_loop(0, (ccnt + _L - 1) // _L, out_group, jnp.int32(0))

    fire(0, 0)
    fire(1, 1)
    total_chunks = 2 * ((_NCHUNK + 1) // 2)

    def step(si, carry):
        for b in range(2):
            k = 2 * si + b
            drain(b)
            extract(k, b)

            @pl.when(k + 2 < total_chunks)
            def _():
                fire(k + 2, b)
        return carry

    lax.fori_loop(0, total_chunks // 2, step, jnp.int32(0))


def _extract_body(u_idx_hbm, i_idx_hbm, ut_hbm, it_hbm, urows_hbm, irows_hbm,
                  idx_v, slab0, slab1, ulist, blist, culist, cblist, rowstage,
                  sem_s0, sem_s1, sem_r):
    c = lax.axis_index("c")
    sub = lax.axis_index("s")

    @pl.when(c == 0)
    def _():
        _extract_pipeline(u_idx_hbm, ut_hbm, urows_hbm, idx_v, slab0, slab1,
                          ulist, blist, culist, cblist, rowstage, sub,
                          sem_s0, sem_s1, sem_r)

    @pl.when(c == 1)
    def _():
        _extract_pipeline(i_idx_hbm, it_hbm, irows_hbm, idx_v, slab0, slab1,
                          ulist, blist, culist, cblist, rowstage, sub,
                          sem_s0, sem_s1, sem_r)


def _dot_body(urows_hbm, irows_hbm, out_hbm,
              ubuf0, ubuf1, ibuf0, ibuf1, accbuf_v, out_v,
              sem_u0, sem_u1, sem_i0, sem_i1):
    wid = lax.axis_index("s") * _NC + lax.axis_index("c")
    base = wid * _BPW
    ubufs, ibufs = (ubuf0, ubuf1), (ibuf0, ibuf1)
    usems, isems = (sem_u0, sem_u1), (sem_i0, sem_i1)
    lanes = lax.iota(jnp.int32, _L)

    def fire(k, b):
        r0 = base + k * _RCH
        pltpu.async_copy(urows_hbm.at[pl.ds(r0, _RCH), :], ubufs[b], usems[b])
        pltpu.async_copy(irows_hbm.at[pl.ds(r0, _RCH), :], ibufs[b], isems[b])

    def drain(b):
        pltpu.make_async_copy(urows_hbm.at[pl.ds(0, _RCH), :], ubufs[b],
                              usems[b]).wait()
        pltpu.make_async_copy(irows_hbm.at[pl.ds(0, _RCH), :], ibufs[b],
                              isems[b]).wait()

    def compute(k, b):
        for g in range(_RCH // _L):
            for j in range(_L):
                r = g * _L + j
                acc = jnp.zeros((_L,), jnp.float32)
                for kk in range(_D // _L):
                    u = ubufs[b][r, pl.ds(kk * _L, _L)]
                    v = ibufs[b][r, pl.ds(kk * _L, _L)]
                    acc = acc + u * v
                accbuf_v[pl.ds(j * _L, _L)] = acc
            s = jnp.zeros((_L,), jnp.float32)
            for col in range(_L):
                s = s + plsc.load_gather(accbuf_v, [lanes * _L + col])
            out_v[pl.ds(k * _RCH + g * _L, _L)] = s

    fire(0, 0)
    fire(1, 1)
    nch = _BPW // _RCH
    for k in range(nch):
        b = k % 2
        drain(b)
        compute(k, b)
        if k + 2 < nch:
            fire(k + 2, b)
    pltpu.sync_copy(out_v, out_hbm.at[pl.ds(base, _BPW)])


@jax.jit
def _run(user_idx, item_idx, user_table_t, item_table_t):
    mesh = plsc.VectorSubcoreMesh(core_axis_name="c", subcore_axis_name="s")
    params = pltpu.CompilerParams(needs_layout_passes=False)

    extract = pl.kernel(
        _extract_body,
        out_type=(jax.ShapeDtypeStruct((_BATCH, _D), jnp.float32),
                  jax.ShapeDtypeStruct((_BATCH, _D), jnp.float32)),
        mesh=mesh,
        compiler_params=params,
        scratch_types=[
            pltpu.VMEM((_BATCH,), jnp.int32),
            pltpu.VMEM((_D, _SLABW), jnp.float32),
            pltpu.VMEM((_D, _SLABW), jnp.float32),
            pltpu.VMEM((_WCAP,), jnp.int32),
            pltpu.VMEM((_WCAP,), jnp.int32),
            pltpu.VMEM((_CCAP,), jnp.int32),
            pltpu.VMEM((_CCAP,), jnp.int32),
            pltpu.VMEM((_L, _D), jnp.float32),
            pltpu.SemaphoreType.DMA,
            pltpu.SemaphoreType.DMA,
            pltpu.SemaphoreType.DMA,
        ],
    )
    urows, irows = extract(user_idx, item_idx, user_table_t, item_table_t)

    dot = pl.kernel(
        _dot_body,
        out_type=jax.ShapeDtypeStruct((_BATCH,), jnp.float32),
        mesh=mesh,
        compiler_params=params,
        scratch_types=[
            pltpu.VMEM((_RCH, _D), jnp.float32),
            pltpu.VMEM((_RCH, _D), jnp.float32),
            pltpu.VMEM((_RCH, _D), jnp.float32),
            pltpu.VMEM((_RCH, _D), jnp.float32),
            pltpu.VMEM((_L * _L,), jnp.float32),
            pltpu.VMEM((_BPW,), jnp.float32),
            pltpu.SemaphoreType.DMA,
            pltpu.SemaphoreType.DMA,
            pltpu.SemaphoreType.DMA,
            pltpu.SemaphoreType.DMA,
        ],
    )
    return dot(urows, irows)


def kernel(user_inputs, item_inputs, user_table, item_table):
    y = _run(user_inputs.reshape(-1), item_inputs.reshape(-1),
             user_table.T, item_table.T)
    return y.reshape(_BATCH, 1)


# speedup vs baseline: 4.1439x; 1.0001x over previous
import functools

import jax
import jax.numpy as jnp
from jax import lax
from jax.experimental import pallas as pl
from jax.experimental.pallas import tpu as pltpu
from jax.experimental.pallas import tpu_sc as plsc

_D = 64
_BATCH = 16384
_V = 1000000

_info = plsc.get_sparse_core_info()
_NC, _NS, _L = _info.num_cores, _info.num_subcores, _info.num_lanes
_NW = _NC * _NS

_TPC = (_V + 127) // 128
_WT = (_TPC + _NS - 1) // _NS
_CT = 7
_IBLK = 4096
_SLABW = _CT * 128
_NCHUNK = (_WT + _CT - 1) // _CT
_MAXCT = _TPC - _CT
_WCAP = 2560
_CCAP = 128

_BPW = _BATCH // _NW
_RCH = 128


def _extract_pipeline(idx_hbm, table_hbm, rows_hbm, idx_v, slab0, slab1,
                      ulist, blist, culist, cblist, rowstage, sub,
                      sem_s0, sem_s1, sem_r):
    slabs = (slab0, slab1)
    sems = (sem_s0, sem_s1)
    wlo = sub * (_WT * 128)
    whi = wlo + _WT * 128
    lanes = lax.iota(jnp.int32, _L)

    cnt = jnp.int32(0)
    for blk in range(_BATCH // _IBLK):
        pltpu.sync_copy(idx_hbm.at[pl.ds(blk * _IBLK, _IBLK)], idx_v)

        def scan_group(g, cnt):
            uvec = idx_v[pl.ds(g * _L, _L)]
            bvec = blk * _IBLK + g * _L + lanes
            mask = (uvec >= wlo) & (uvec < whi)
            plsc.store_compressed(ulist.at[pl.ds(cnt, _L)], uvec, mask=mask)
            plsc.store_compressed(blist.at[pl.ds(cnt, _L)], bvec, mask=mask)
            c16 = plsc.all_reduce_population_count(mask)
            return cnt + c16[0]

        cnt = lax.fori_loop(0, _IBLK // _L, scan_group, cnt)
    nsel_groups = (cnt + _L - 1) // _L

    def slab_start(k):
        ct = jnp.minimum(sub * _WT + k * _CT, _MAXCT)
        return pl.multiple_of(ct * 128, 128)

    def fire(k, b):
        pltpu.async_copy(table_hbm.at[:, pl.ds(slab_start(k), _SLABW)],
                         slabs[b], sems[b])

    def drain(b):
        pltpu.make_async_copy(
            table_hbm.at[:, pl.ds(pl.multiple_of(jnp.int32(0), 128), _SLABW)],
            slabs[b], sems[b]).wait()

    def extract(k, b):
        clo = slab_start(k)
        chi = jnp.minimum(clo + _SLABW, whi)

        def match_group(g, ccnt):
            uvec = ulist[pl.ds(g * _L, _L)]
            bvec = blist[pl.ds(g * _L, _L)]
            valid = (g * _L + lanes) < cnt
            mask = valid & (uvec >= clo) & (uvec < chi)
            plsc.store_compressed(culist.at[pl.ds(ccnt, _L)], uvec - clo,
                                  mask=mask)
            plsc.store_compressed(cblist.at[pl.ds(ccnt, _L)], bvec, mask=mask)
            c16 = plsc.all_reduce_population_count(mask)
            return ccnt + c16[0]

        ccnt = lax.fori_loop(0, nsel_groups, match_group, jnp.int32(0))

        def out_group(g2, carry):
            cu = culist[pl.ds(g2 * _L, _L)]
            cb = cblist[pl.ds(g2 * _L, _L)]
            for j in range(_L):
                @pl.when(g2 * _L + j < ccnt)
                def _():
                    colv = jnp.full((_L,), cu[j], jnp.int32)
                    for dg in range(_D // _L):
                        col = plsc.load_gather(slabs[b],
                                               [dg * _L + lanes, colv])
                        rowstage[j, pl.ds(dg * _L, _L)] = col
                    pltpu.async_copy(rowstage.at[j], rows_hbm.at[cb[j]],
                                     sem_r)
            for j in range(_L):
                @pl.when(g2 * _L + j < ccnt)
                def _():
                    pltpu.make_async_copy(rowstage.at[j], rows_hbm.at[cb[j]],
                                          sem_r).wait()
            return carry

        lax.fori_loop(0, (ccnt + _L - 1) // _L, out_group, jnp.int32(0))

    fire(0, 0)
    fire(1, 1)
    total_chunks = 2 * ((_NCHUNK + 1) // 2)

    def step(si, carry):
        for b in range(2):
            k = 2 * si + b
            drain(b)
            extract(k, b)

            @pl.when(k + 2 < total_chunks)
            def _():
                fire(k + 2, b)
        return carry

    lax.fori_loop(0, total_chunks // 2, step, jnp.int32(0))


def _extract_body(u_idx_hbm, i_idx_hbm, ut_hbm, it_hbm, urows_hbm, irows_hbm,
                  idx_v, slab0, slab1, ulist, blist, culist, cblist, rowstage,
                  sem_s0, sem_s1, sem_r):
    c = lax.axis_index("c")
    sub = lax.axis_index("s")

    @pl.when(c == 0)
    def _():
        _extract_pipeline(u_idx_hbm, ut_hbm, urows_hbm, idx_v, slab0, slab1,
                          ulist, blist, culist, cblist, rowstage, sub,
                          sem_s0, sem_s1, sem_r)

    @pl.when(c == 1)
    def _():
        _extract_pipeline(i_idx_hbm, it_hbm, irows_hbm, idx_v, slab0, slab1,
                          ulist, blist, culist, cblist, rowstage, sub,
                          sem_s0, sem_s1, sem_r)


def _dot_body(urows_hbm, irows_hbm, out_hbm,
              ubuf0, ubuf1, ibuf0, ibuf1, accbuf_v, out_v,
              sem_u0, sem_u1, sem_i0, sem_i1):
    wid = lax.axis_index("s") * _NC + lax.axis_index("c")
    base = wid * _BPW
    ubufs, ibufs = (ubuf0, ubuf1), (ibuf0, ibuf1)
    usems, isems = (sem_u0, sem_u1), (sem_i0, sem_i1)
    lanes = lax.iota(jnp.int32, _L)

    def fire(k, b):
        r0 = base + k * _RCH
        pltpu.async_copy(urows_hbm.at[pl.ds(r0, _RCH), :], ubufs[b], usems[b])
        pltpu.async_copy(irows_hbm.at[pl.ds(r0, _RCH), :], ibufs[b], isems[b])

    def drain(b):
        pltpu.make_async_copy(urows_hbm.at[pl.ds(0, _RCH), :], ubufs[b],
                              usems[b]).wait()
        pltpu.make_async_copy(irows_hbm.at[pl.ds(0, _RCH), :], ibufs[b],
                              isems[b]).wait()

    def compute(k, b):
        for g in range(_RCH // _L):
            for j in range(_L):
                r = g * _L + j
                acc = jnp.zeros((_L,), jnp.float32)
                for kk in range(_D // _L):
                    u = ubufs[b][r, pl.ds(kk * _L, _L)]
                    v = ibufs[b][r, pl.ds(kk * _L, _L)]
                    acc = acc + u * v
                accbuf_v[pl.ds(j * _L, _L)] = acc
            s = jnp.zeros((_L,), jnp.float32)
            for col in range(_L):
                s = s + plsc.load_gather(accbuf_v, [lanes * _L + col])
            out_v[pl.ds(k * _RCH + g * _L, _L)] = s

    fire(0, 0)
    fire(1, 1)
    nch = _BPW // _RCH
    for k in range(nch):
        b = k % 2
        drain(b)
        compute(k, b)
        if k + 2 < nch:
            fire(k + 2, b)
    pltpu.sync_copy(out_v, out_hbm.at[pl.ds(base, _BPW)])


@jax.jit
def _run(user_idx, item_idx, user_table_t, item_table_t):
    mesh = plsc.VectorSubcoreMesh(core_axis_name="c", subcore_axis_name="s")
    params = pltpu.CompilerParams(needs_layout_passes=False)

    extract = pl.kernel(
        _extract_body,
        out_type=(jax.ShapeDtypeStruct((_BATCH, _D), jnp.float32),
                  jax.ShapeDtypeStruct((_BATCH, _D), jnp.float32)),
        mesh=mesh,
        compiler_params=params,
        scratch_types=[
            pltpu.VMEM((_IBLK,), jnp.int32),
            pltpu.VMEM((_D, _SLABW), jnp.float32),
            pltpu.VMEM((_D, _SLABW), jnp.float32),
            pltpu.VMEM((_WCAP,), jnp.int32),
            pltpu.VMEM((_WCAP,), jnp.int32),
            pltpu.VMEM((_CCAP,), jnp.int32),
            pltpu.VMEM((_CCAP,), jnp.int32),
            pltpu.VMEM((_L, _D), jnp.float32),
            pltpu.SemaphoreType.DMA,
            pltpu.SemaphoreType.DMA,
            pltpu.SemaphoreType.DMA,
        ],
    )
    urows, irows = extract(user_idx, item_idx, user_table_t, item_table_t)

    dot = pl.kernel(
        _dot_body,
        out_type=jax.ShapeDtypeStruct((_BATCH,), jnp.float32),
        mesh=mesh,
        compiler_params=params,
        scratch_types=[
            pltpu.VMEM((_RCH, _D), jnp.float32),
            pltpu.VMEM((_RCH, _D), jnp.float32),
            pltpu.VMEM((_RCH, _D), jnp.float32),
            pltpu.VMEM((_RCH, _D), jnp.float32),
            pltpu.VMEM((_L * _L,), jnp.float32),
            pltpu.VMEM((_BPW,), jnp.float32),
            pltpu.SemaphoreType.DMA,
            pltpu.SemaphoreType.DMA,
            pltpu.SemaphoreType.DMA,
            pltpu.SemaphoreType.DMA,
        ],
    )
    return dot(urows, irows)


def kernel(user_inputs, item_inputs, user_table, item_table):
    y = _run(user_inputs.reshape(-1), item_inputs.reshape(-1),
             user_table.T, item_table.T)
    return y.reshape(_BATCH, 1)
